# baseline (speedup 1.0000x reference)
import functools
import jax
import jax.numpy as jnp
from jax import lax
from jax.experimental import pallas as pl
from jax.experimental.pallas import tpu as pltpu
from jax.experimental.pallas import tpu_sc as plsc

L = 2048
D_MODEL = 1024
N_HEADS = 16
DH = 64
D_FF = 2048
SAMPLE_K = 40
N_TOP = 40
BLK = 256
NEG = -3.0e38

NW = 32
ROWS_W = L // NW
ROWS_CH = 32
N_CH = ROWS_W // ROWS_CH


def _count_sc(half, idx_hbm, z_hbm, c_hbm, idx_v, buf):
    cid = lax.axis_index("c")
    sid = lax.axis_index("s")
    wid = sid * 2 + cid
    row0 = half * (L // 2) + wid * ROWS_CH
    pltpu.sync_copy(idx_hbm.at[pl.ds(row0 * SAMPLE_K, ROWS_CH * SAMPLE_K)],
                    idx_v)
    pltpu.sync_copy(z_hbm, buf)
    lane = lax.iota(jnp.int32, 16)
    lane40 = lane * SAMPLE_K
    ones = jnp.full((16,), 1.0, jnp.float32)
    for s in range(SAMPLE_K):
        for g in range(ROWS_CH // 16):
            gidx = lane40 + ((g * 16) * SAMPLE_K + s)
            col = plsc.load_gather(idx_v, [gidx])
            flat = (g * 16 + lane) * L + col
            cur = plsc.load_gather(buf, [flat])
            plsc.store_scatter(buf, [flat], cur + ones)
    pltpu.sync_copy(buf, c_hbm.at[pl.ds(wid * ROWS_CH * L, ROWS_CH * L)])


def _m_kernel(c_ref, tgt_ref, src_ref, m_ref):
    cnt = c_ref[...]
    mask = cnt > 0.0
    ks = lax.dot_general(cnt, src_ref[...], (((1,), (0,)), ((), ())),
                         preferred_element_type=jnp.float32)
    qks = tgt_ref[...] * ks
    for h in range(N_HEADS):
        q = tgt_ref[:, h * DH:(h + 1) * DH]
        k = src_ref[:, h * DH:(h + 1) * DH]
        s_blk = lax.dot_general(q, k, (((1,), (1,)), ((), ())),
                                preferred_element_type=jnp.float32)
        msum = jnp.sum(qks[:, h * DH:(h + 1) * DH], axis=1)
        mmax = jnp.max(jnp.where(mask, s_blk, NEG), axis=1)
        m_ref[h, :] = mmax - msum * (1.0 / L)


def _topk_kernel(mlo_ref, mhi_ref, top_ref):
    mv = jnp.concatenate([mlo_ref[...], mhi_ref[...]], axis=1)
    iota_k = lax.broadcasted_iota(jnp.int32, (N_HEADS, L), 1)
    cols = []
    for _ in range(N_TOP):
        cur = jnp.max(mv, axis=1, keepdims=True)
        am = jnp.min(jnp.where(mv == cur, iota_k, L), axis=1)
        cols.append(am)
        mv = jnp.where(iota_k == am[:, None], NEG, mv)
    top_ref[...] = jnp.stack(cols, axis=1)


def _attn_kernel(top_ref, tgt_ref, src_ref, att_ref):
    i = pl.program_id(0)
    iota_l = lax.broadcasted_iota(jnp.int32, (L, N_TOP), 0)
    for hh in range(2):
        mt = top_ref[pl.ds(i * 2 + hh, 1), :]
        oht = (iota_l == mt).astype(jnp.float32)
        q_h = tgt_ref[:, hh * DH:(hh + 1) * DH]
        k_h = src_ref[:, hh * DH:(hh + 1) * DH]
        q_red = lax.dot_general(oht, q_h, (((0,), (0,)), ((), ())),
                                preferred_element_type=jnp.float32)
        scores = lax.dot_general(q_red, k_h, (((1,), (1,)), ((), ())),
                                 preferred_element_type=jnp.float32) * 0.125
        smax = jnp.max(scores, axis=1, keepdims=True)
        e = jnp.exp(scores - smax)
        attn = e / jnp.sum(e, axis=1, keepdims=True)
        upd = lax.dot_general(attn, k_h, (((1,), (0,)), ((), ())),
                              preferred_element_type=jnp.float32)
        mean_v = jnp.sum(k_h, axis=0, keepdims=True) * (1.0 / L)
        ind = jnp.sum(oht, axis=1, keepdims=True)
        att_ref[:, hh * DH:(hh + 1) * DH] = (1.0 - ind) * mean_v + lax.dot_general(
            oht, upd, (((1,), (0,)), ((), ())), preferred_element_type=jnp.float32)


def _ln(x, g, b):
    mu = jnp.mean(x, axis=1, keepdims=True)
    var = jnp.mean((x - mu) ** 2, axis=1, keepdims=True)
    return (x - mu) * lax.rsqrt(var + 1e-5) * g + b


def _ffn_kernel(tgt_ref, att_ref, w1_ref, b1_ref, w2_ref, b2_ref,
                g1_ref, be1_ref, g2_ref, be2_ref, out_ref):
    skipped = tgt_ref[...] + att_ref[...]
    normed = _ln(skipped, g1_ref[...], be1_ref[...])
    h1 = lax.dot_general(normed, w1_ref[...], (((1,), (1,)), ((), ())),
                         preferred_element_type=jnp.float32) + b1_ref[...]
    h1 = jnp.maximum(h1, 0.0)
    proj = lax.dot_general(h1, w2_ref[...], (((1,), (1,)), ((), ())),
                           preferred_element_type=jnp.float32) + b2_ref[...]
    out_ref[...] = _ln(normed + proj, g2_ref[...], be2_ref[...])


def kernel(target, source, W1, b1, W2, b2, g1, be1, g2, be2, index_sample):
    tgt = target.reshape(L, D_MODEL)
    src = source.reshape(L, D_MODEL)
    idx = index_sample.astype(jnp.int32)

    idx_flat = idx.reshape(L * SAMPLE_K)
    zblk = jnp.zeros((ROWS_CH * L,), jnp.float32)

    mesh = plsc.VectorSubcoreMesh(core_axis_name="c", subcore_axis_name="s")
    c_halves = []
    for half in range(2):
        c_h = pl.kernel(
            functools.partial(_count_sc, half),
            out_type=jax.ShapeDtypeStruct((L // 2 * L,), jnp.float32),
            mesh=mesh,
            scratch_types=[
                pltpu.VMEM((ROWS_CH * SAMPLE_K,), jnp.int32),
                pltpu.VMEM((ROWS_CH * L,), jnp.float32),
            ],
            compiler_params=pltpu.CompilerParams(needs_layout_passes=False),
        )(idx_flat, zblk)
        c_halves.append(c_h.reshape(L // 2, L))

    m_halves = []
    for half in range(2):
        m_h = pl.pallas_call(
            _m_kernel,
            grid=(L // 2 // BLK,),
            in_specs=[
                pl.BlockSpec((BLK, L), lambda b: (b, 0)),
                pl.BlockSpec((BLK, D_MODEL),
                             lambda b, H=half: (b + H * (L // 2 // BLK), 0)),
                pl.BlockSpec((L, D_MODEL), lambda b: (0, 0)),
            ],
            out_specs=pl.BlockSpec((N_HEADS, BLK), lambda b: (0, b)),
            out_shape=jax.ShapeDtypeStruct((N_HEADS, L // 2), jnp.float32),
        )(c_halves[half], tgt, src)
        m_halves.append(m_h)

    m_top = pl.pallas_call(
        _topk_kernel,
        out_shape=jax.ShapeDtypeStruct((N_HEADS, N_TOP), jnp.int32),
    )(m_halves[0], m_halves[1])

    attended = pl.pallas_call(
        _attn_kernel,
        grid=(N_HEADS // 2,),
        in_specs=[
            pl.BlockSpec((N_HEADS, N_TOP), lambda h: (0, 0)),
            pl.BlockSpec((L, 2 * DH), lambda h: (0, h)),
            pl.BlockSpec((L, 2 * DH), lambda h: (0, h)),
        ],
        out_specs=pl.BlockSpec((L, 2 * DH), lambda h: (0, h)),
        out_shape=jax.ShapeDtypeStruct((L, D_MODEL), jnp.float32),
    )(m_top, tgt, src)

    out = pl.pallas_call(
        _ffn_kernel,
        grid=(L // BLK,),
        in_specs=[
            pl.BlockSpec((BLK, D_MODEL), lambda b: (b, 0)),
            pl.BlockSpec((BLK, D_MODEL), lambda b: (b, 0)),
            pl.BlockSpec((D_FF, D_MODEL), lambda b: (0, 0)),
            pl.BlockSpec((1, D_FF), lambda b: (0, 0)),
            pl.BlockSpec((D_MODEL, D_FF), lambda b: (0, 0)),
            pl.BlockSpec((1, D_MODEL), lambda b: (0, 0)),
            pl.BlockSpec((1, D_MODEL), lambda b: (0, 0)),
            pl.BlockSpec((1, D_MODEL), lambda b: (0, 0)),
            pl.BlockSpec((1, D_MODEL), lambda b: (0, 0)),
            pl.BlockSpec((1, D_MODEL), lambda b: (0, 0)),
        ],
        out_specs=pl.BlockSpec((BLK, D_MODEL), lambda b: (b, 0)),
        out_shape=jax.ShapeDtypeStruct((L, D_MODEL), jnp.float32),
    )(tgt, attended, W1, b1.reshape(1, D_FF), W2, b2.reshape(1, D_MODEL),
      g1.reshape(1, D_MODEL), be1.reshape(1, D_MODEL),
      g2.reshape(1, D_MODEL), be2.reshape(1, D_MODEL))

    return out.reshape(L, 1, D_MODEL)


# speedup vs baseline: 1.0708x; 1.0708x over previous
import functools
import jax
import jax.numpy as jnp
from jax import lax
from jax.experimental import pallas as pl
from jax.experimental.pallas import tpu as pltpu
from jax.experimental.pallas import tpu_sc as plsc

L = 2048
D_MODEL = 1024
N_HEADS = 16
DH = 64
D_FF = 2048
SAMPLE_K = 40
N_TOP = 40
BLK = 512
NEG = -3.0e38

NW = 32
ROWS_W = L // NW
ROWS_CH = 32
N_CH = ROWS_W // ROWS_CH


def _count_sc(idx_hbm, z_hbm, c_hbm, idx_v, buf):
    cid = lax.axis_index("c")
    sid = lax.axis_index("s")
    wid = sid * 2 + cid
    pltpu.sync_copy(idx_hbm.at[pl.ds(wid * ROWS_W * SAMPLE_K,
                                     ROWS_W * SAMPLE_K)], idx_v)
    pltpu.sync_copy(z_hbm, buf)
    lane = lax.iota(jnp.int32, 16)
    lane40 = lane * SAMPLE_K
    ones = jnp.full((16,), 1.0, jnp.float32)
    zeros = jnp.zeros((16,), jnp.float32)
    for ch in range(N_CH):
        for s in range(SAMPLE_K):
            for g in range(ROWS_CH // 16):
                gidx = lane40 + ((ch * ROWS_CH + g * 16) * SAMPLE_K + s)
                col = plsc.load_gather(idx_v, [gidx])
                flat = (g * 16 + lane) * L + col
                cur = plsc.load_gather(buf, [flat])
                plsc.store_scatter(buf, [flat], cur + ones)
        pltpu.sync_copy(
            buf, c_hbm.at[pl.ds((wid * ROWS_W + ch * ROWS_CH) * L, ROWS_CH * L)])
        if ch + 1 < N_CH:
            for s in range(SAMPLE_K):
                for g in range(ROWS_CH // 16):
                    gidx = lane40 + ((ch * ROWS_CH + g * 16) * SAMPLE_K + s)
                    col = plsc.load_gather(idx_v, [gidx])
                    flat = (g * 16 + lane) * L + col
                    plsc.store_scatter(buf, [flat], zeros)


def _m_kernel(c_ref, tgt_ref, src_ref, m_ref):
    cnt = c_ref[...]
    mask = cnt > 0.0
    ks = lax.dot_general(cnt, src_ref[...], (((1,), (0,)), ((), ())),
                         preferred_element_type=jnp.float32)
    qks = tgt_ref[...] * ks
    for h in range(N_HEADS):
        q = tgt_ref[:, h * DH:(h + 1) * DH]
        k = src_ref[:, h * DH:(h + 1) * DH]
        s_blk = lax.dot_general(q, k, (((1,), (1,)), ((), ())),
                                preferred_element_type=jnp.float32)
        msum = jnp.sum(qks[:, h * DH:(h + 1) * DH], axis=1)
        mmax = jnp.max(jnp.where(mask, s_blk, NEG), axis=1)
        m_ref[h, :] = mmax - msum * (1.0 / L)


def _topk_kernel(m_ref, top_ref):
    mv = m_ref[...]
    iota_k = lax.broadcasted_iota(jnp.int32, (N_HEADS, L), 1)
    cols = []
    for _ in range(N_TOP):
        cur = jnp.max(mv, axis=1, keepdims=True)
        am = jnp.min(jnp.where(mv == cur, iota_k, L), axis=1)
        cols.append(am)
        mv = jnp.where(iota_k == am[:, None], NEG, mv)
    top_ref[...] = jnp.stack(cols, axis=1)


def _attn_kernel(top_ref, tgt_ref, src_ref, att_ref):
    i = pl.program_id(0)
    iota_l = lax.broadcasted_iota(jnp.int32, (L, N_TOP), 0)
    for hh in range(2):
        mt = top_ref[pl.ds(i * 2 + hh, 1), :]
        oht = (iota_l == mt).astype(jnp.float32)
        q_h = tgt_ref[:, hh * DH:(hh + 1) * DH]
        k_h = src_ref[:, hh * DH:(hh + 1) * DH]
        q_red = lax.dot_general(oht, q_h, (((0,), (0,)), ((), ())),
                                preferred_element_type=jnp.float32)
        scores = lax.dot_general(q_red, k_h, (((1,), (1,)), ((), ())),
                                 preferred_element_type=jnp.float32) * 0.125
        smax = jnp.max(scores, axis=1, keepdims=True)
        e = jnp.exp(scores - smax)
        attn = e / jnp.sum(e, axis=1, keepdims=True)
        upd = lax.dot_general(attn, k_h, (((1,), (0,)), ((), ())),
                              preferred_element_type=jnp.float32)
        mean_v = jnp.sum(k_h, axis=0, keepdims=True) * (1.0 / L)
        ind = jnp.sum(oht, axis=1, keepdims=True)
        att_ref[:, hh * DH:(hh + 1) * DH] = (1.0 - ind) * mean_v + lax.dot_general(
            oht, upd, (((1,), (0,)), ((), ())), preferred_element_type=jnp.float32)


def _ln(x, g, b):
    mu = jnp.mean(x, axis=1, keepdims=True)
    var = jnp.mean((x - mu) ** 2, axis=1, keepdims=True)
    return (x - mu) * lax.rsqrt(var + 1e-5) * g + b


def _ffn_kernel(tgt_ref, att_ref, w1_ref, b1_ref, w2_ref, b2_ref,
                g1_ref, be1_ref, g2_ref, be2_ref, out_ref):
    skipped = tgt_ref[...] + att_ref[...]
    normed = _ln(skipped, g1_ref[...], be1_ref[...])
    h1 = lax.dot_general(normed, w1_ref[...], (((1,), (1,)), ((), ())),
                         preferred_element_type=jnp.float32) + b1_ref[...]
    h1 = jnp.maximum(h1, 0.0)
    proj = lax.dot_general(h1, w2_ref[...], (((1,), (1,)), ((), ())),
                           preferred_element_type=jnp.float32) + b2_ref[...]
    out_ref[...] = _ln(normed + proj, g2_ref[...], be2_ref[...])


def kernel(target, source, W1, b1, W2, b2, g1, be1, g2, be2, index_sample):
    tgt = target.reshape(L, D_MODEL)
    src = source.reshape(L, D_MODEL)
    idx = index_sample.astype(jnp.int32)

    idx_flat = idx.reshape(L * SAMPLE_K)
    zblk = jnp.zeros((ROWS_CH * L,), jnp.float32)

    mesh = plsc.VectorSubcoreMesh(core_axis_name="c", subcore_axis_name="s")
    cmat = pl.kernel(
        _count_sc,
        out_type=jax.ShapeDtypeStruct((L * L,), jnp.float32),
        mesh=mesh,
        scratch_types=[
            pltpu.VMEM((SAMPLE_K * ROWS_W,), jnp.int32),
            pltpu.VMEM((ROWS_CH * L,), jnp.float32),
        ],
        compiler_params=pltpu.CompilerParams(needs_layout_passes=False),
    )(idx_flat, zblk)
    cmat = cmat.reshape(L, L)

    m = pl.pallas_call(
        _m_kernel,
        grid=(L // BLK,),
        in_specs=[
            pl.BlockSpec((BLK, L), lambda b: (b, 0)),
            pl.BlockSpec((BLK, D_MODEL), lambda b: (b, 0)),
            pl.BlockSpec((L, D_MODEL), lambda b: (0, 0)),
        ],
        out_specs=pl.BlockSpec((N_HEADS, BLK), lambda b: (0, b)),
        out_shape=jax.ShapeDtypeStruct((N_HEADS, L), jnp.float32),
    )(cmat, tgt, src)

    m_top = pl.pallas_call(
        _topk_kernel,
        out_shape=jax.ShapeDtypeStruct((N_HEADS, N_TOP), jnp.int32),
    )(m)

    attended = pl.pallas_call(
        _attn_kernel,
        grid=(N_HEADS // 2,),
        in_specs=[
            pl.BlockSpec((N_HEADS, N_TOP), lambda h: (0, 0)),
            pl.BlockSpec((L, 2 * DH), lambda h: (0, h)),
            pl.BlockSpec((L, 2 * DH), lambda h: (0, h)),
        ],
        out_specs=pl.BlockSpec((L, 2 * DH), lambda h: (0, h)),
        out_shape=jax.ShapeDtypeStruct((L, D_MODEL), jnp.float32),
    )(m_top, tgt, src)

    out = pl.pallas_call(
        _ffn_kernel,
        grid=(L // BLK,),
        in_specs=[
            pl.BlockSpec((BLK, D_MODEL), lambda b: (b, 0)),
            pl.BlockSpec((BLK, D_MODEL), lambda b: (b, 0)),
            pl.BlockSpec((D_FF, D_MODEL), lambda b: (0, 0)),
            pl.BlockSpec((1, D_FF), lambda b: (0, 0)),
            pl.BlockSpec((D_MODEL, D_FF), lambda b: (0, 0)),
            pl.BlockSpec((1, D_MODEL), lambda b: (0, 0)),
            pl.BlockSpec((1, D_MODEL), lambda b: (0, 0)),
            pl.BlockSpec((1, D_MODEL), lambda b: (0, 0)),
            pl.BlockSpec((1, D_MODEL), lambda b: (0, 0)),
            pl.BlockSpec((1, D_MODEL), lambda b: (0, 0)),
        ],
        out_specs=pl.BlockSpec((BLK, D_MODEL), lambda b: (b, 0)),
        out_shape=jax.ShapeDtypeStruct((L, D_MODEL), jnp.float32),
    )(tgt, attended, W1, b1.reshape(1, D_FF), W2, b2.reshape(1, D_MODEL),
      g1.reshape(1, D_MODEL), be1.reshape(1, D_MODEL),
      g2.reshape(1, D_MODEL), be2.reshape(1, D_MODEL))

    return out.reshape(L, 1, D_MODEL)


# speedup vs baseline: 1.0771x; 1.0059x over previous
import functools
import jax
import jax.numpy as jnp
from jax import lax
from jax.experimental import pallas as pl
from jax.experimental.pallas import tpu as pltpu
from jax.experimental.pallas import tpu_sc as plsc

L = 2048
D_MODEL = 1024
N_HEADS = 16
DH = 64
D_FF = 2048
SAMPLE_K = 40
N_TOP = 40
BLK = 512
NEG = -3.0e38

NW = 32
ROWS_W = L // NW
ROWS_CH = 32
N_CH = ROWS_W // ROWS_CH


def _count_sc(idx_hbm, z_hbm, c_hbm, idx_v, buf):
    cid = lax.axis_index("c")
    sid = lax.axis_index("s")
    wid = sid * 2 + cid
    pltpu.sync_copy(idx_hbm.at[pl.ds(wid * ROWS_W * SAMPLE_K,
                                     ROWS_W * SAMPLE_K)], idx_v)
    pltpu.sync_copy(z_hbm, buf)
    lane = lax.iota(jnp.int32, 16)
    lane40 = lane * SAMPLE_K
    ones = jnp.full((16,), 1.0, jnp.float32)
    zeros = jnp.zeros((16,), jnp.float32)
    for ch in range(N_CH):
        for s in range(SAMPLE_K):
            for g in range(ROWS_CH // 16):
                gidx = lane40 + ((ch * ROWS_CH + g * 16) * SAMPLE_K + s)
                col = plsc.load_gather(idx_v, [gidx])
                flat = (g * 16 + lane) * L + col
                cur = plsc.load_gather(buf, [flat])
                plsc.store_scatter(buf, [flat], cur + ones)
        pltpu.sync_copy(
            buf, c_hbm.at[pl.ds((wid * ROWS_W + ch * ROWS_CH) * L, ROWS_CH * L)])
        if ch + 1 < N_CH:
            for s in range(SAMPLE_K):
                for g in range(ROWS_CH // 16):
                    gidx = lane40 + ((ch * ROWS_CH + g * 16) * SAMPLE_K + s)
                    col = plsc.load_gather(idx_v, [gidx])
                    flat = (g * 16 + lane) * L + col
                    plsc.store_scatter(buf, [flat], zeros)


def _m_kernel(c_ref, tgt_ref, src_ref, m_ref):
    cnt = c_ref[...]
    mask = cnt > 0.0
    ks = lax.dot_general(cnt, src_ref[...], (((1,), (0,)), ((), ())),
                         preferred_element_type=jnp.float32)
    qks = tgt_ref[...] * ks
    for h in range(N_HEADS):
        q = tgt_ref[:, h * DH:(h + 1) * DH]
        k = src_ref[:, h * DH:(h + 1) * DH]
        s_blk = lax.dot_general(q, k, (((1,), (1,)), ((), ())),
                                preferred_element_type=jnp.float32)
        msum = jnp.sum(qks[:, h * DH:(h + 1) * DH], axis=1)
        mmax = jnp.max(jnp.where(mask, s_blk, NEG), axis=1)
        m_ref[h, :] = mmax - msum * (1.0 / L)


def _attn_kernel(m_ref, tgt_ref, src_ref, att_ref, top_ref):
    i = pl.program_id(0)

    @pl.when(i == 0)
    def _compute_topk():
        mv = m_ref[...]
        iota_k = lax.broadcasted_iota(jnp.int32, (N_HEADS, L), 1)
        cols = []
        for _ in range(N_TOP):
            cur = jnp.max(mv, axis=1, keepdims=True)
            am = jnp.min(jnp.where(mv == cur, iota_k, L), axis=1)
            cols.append(am)
            mv = jnp.where(iota_k == am[:, None], NEG, mv)
        top_ref[...] = jnp.stack(cols, axis=1)

    iota_l = lax.broadcasted_iota(jnp.int32, (L, N_TOP), 0)
    for hh in range(2):
        mt = top_ref[pl.ds(i * 2 + hh, 1), :]
        oht = (iota_l == mt).astype(jnp.float32)
        q_h = tgt_ref[:, hh * DH:(hh + 1) * DH]
        k_h = src_ref[:, hh * DH:(hh + 1) * DH]
        q_red = lax.dot_general(oht, q_h, (((0,), (0,)), ((), ())),
                                preferred_element_type=jnp.float32)
        scores = lax.dot_general(q_red, k_h, (((1,), (1,)), ((), ())),
                                 preferred_element_type=jnp.float32) * 0.125
        smax = jnp.max(scores, axis=1, keepdims=True)
        e = jnp.exp(scores - smax)
        attn = e / jnp.sum(e, axis=1, keepdims=True)
        upd = lax.dot_general(attn, k_h, (((1,), (0,)), ((), ())),
                              preferred_element_type=jnp.float32)
        mean_v = jnp.sum(k_h, axis=0, keepdims=True) * (1.0 / L)
        ind = jnp.sum(oht, axis=1, keepdims=True)
        att_ref[:, hh * DH:(hh + 1) * DH] = (1.0 - ind) * mean_v + lax.dot_general(
            oht, upd, (((1,), (0,)), ((), ())), preferred_element_type=jnp.float32)


def _ln(x, g, b):
    mu = jnp.mean(x, axis=1, keepdims=True)
    var = jnp.mean((x - mu) ** 2, axis=1, keepdims=True)
    return (x - mu) * lax.rsqrt(var + 1e-5) * g + b


def _ffn_kernel(tgt_ref, att_ref, w1_ref, b1_ref, w2_ref, b2_ref,
                g1_ref, be1_ref, g2_ref, be2_ref, out_ref):
    skipped = tgt_ref[...] + att_ref[...]
    normed = _ln(skipped, g1_ref[...], be1_ref[...])
    h1 = lax.dot_general(normed, w1_ref[...], (((1,), (1,)), ((), ())),
                         preferred_element_type=jnp.float32) + b1_ref[...]
    h1 = jnp.maximum(h1, 0.0)
    proj = lax.dot_general(h1, w2_ref[...], (((1,), (1,)), ((), ())),
                           preferred_element_type=jnp.float32) + b2_ref[...]
    out_ref[...] = _ln(normed + proj, g2_ref[...], be2_ref[...])


def kernel(target, source, W1, b1, W2, b2, g1, be1, g2, be2, index_sample):
    tgt = target.reshape(L, D_MODEL)
    src = source.reshape(L, D_MODEL)
    idx = index_sample.astype(jnp.int32)

    idx_flat = idx.reshape(L * SAMPLE_K)
    zblk = jnp.zeros((ROWS_CH * L,), jnp.float32)

    mesh = plsc.VectorSubcoreMesh(core_axis_name="c", subcore_axis_name="s")
    cmat = pl.kernel(
        _count_sc,
        out_type=jax.ShapeDtypeStruct((L * L,), jnp.float32),
        mesh=mesh,
        scratch_types=[
            pltpu.VMEM((SAMPLE_K * ROWS_W,), jnp.int32),
            pltpu.VMEM((ROWS_CH * L,), jnp.float32),
        ],
        compiler_params=pltpu.CompilerParams(needs_layout_passes=False),
    )(idx_flat, zblk)
    cmat = cmat.reshape(L, L)

    m = pl.pallas_call(
        _m_kernel,
        grid=(L // BLK,),
        in_specs=[
            pl.BlockSpec((BLK, L), lambda b: (b, 0)),
            pl.BlockSpec((BLK, D_MODEL), lambda b: (b, 0)),
            pl.BlockSpec((L, D_MODEL), lambda b: (0, 0)),
        ],
        out_specs=pl.BlockSpec((N_HEADS, BLK), lambda b: (0, b)),
        out_shape=jax.ShapeDtypeStruct((N_HEADS, L), jnp.float32),
    )(cmat, tgt, src)

    attended = pl.pallas_call(
        _attn_kernel,
        grid=(N_HEADS // 2,),
        in_specs=[
            pl.BlockSpec((N_HEADS, L), lambda h: (0, 0)),
            pl.BlockSpec((L, 2 * DH), lambda h: (0, h)),
            pl.BlockSpec((L, 2 * DH), lambda h: (0, h)),
        ],
        out_specs=pl.BlockSpec((L, 2 * DH), lambda h: (0, h)),
        out_shape=jax.ShapeDtypeStruct((L, D_MODEL), jnp.float32),
        scratch_shapes=[pltpu.VMEM((N_HEADS, N_TOP), jnp.int32)],
    )(m, tgt, src)

    out = pl.pallas_call(
        _ffn_kernel,
        grid=(L // BLK,),
        in_specs=[
            pl.BlockSpec((BLK, D_MODEL), lambda b: (b, 0)),
            pl.BlockSpec((BLK, D_MODEL), lambda b: (b, 0)),
            pl.BlockSpec((D_FF, D_MODEL), lambda b: (0, 0)),
            pl.BlockSpec((1, D_FF), lambda b: (0, 0)),
            pl.BlockSpec((D_MODEL, D_FF), lambda b: (0, 0)),
            pl.BlockSpec((1, D_MODEL), lambda b: (0, 0)),
            pl.BlockSpec((1, D_MODEL), lambda b: (0, 0)),
            pl.BlockSpec((1, D_MODEL), lambda b: (0, 0)),
            pl.BlockSpec((1, D_MODEL), lambda b: (0, 0)),
            pl.BlockSpec((1, D_MODEL), lambda b: (0, 0)),
        ],
        out_specs=pl.BlockSpec((BLK, D_MODEL), lambda b: (b, 0)),
        out_shape=jax.ShapeDtypeStruct((L, D_MODEL), jnp.float32),
    )(tgt, attended, W1, b1.reshape(1, D_FF), W2, b2.reshape(1, D_MODEL),
      g1.reshape(1, D_MODEL), be1.reshape(1, D_MODEL),
      g2.reshape(1, D_MODEL), be2.reshape(1, D_MODEL))

    return out.reshape(L, 1, D_MODEL)
